# baseline (speedup 1.0000x reference)
import functools

import jax
import jax.numpy as jnp
from jax import lax
from jax.experimental import pallas as pl
from jax.experimental.pallas import tpu as pltpu
from jax.experimental.pallas import tpu_sc as plsc

F32 = jnp.float32
NC = 2
NS = 16
BLK = 128
CW = 128


def _mesh():
    return plsc.VectorSubcoreMesh(core_axis_name="c", subcore_axis_name="s",
                                  num_cores=NC, num_subcores=NS)


def _copy_rows(src_ref, dst_ref, s, n_rows, add=False):
    main = (n_rows // NS) & ~7
    rem = n_rows - NS * main
    pltpu.sync_copy(src_ref.at[pl.ds(s * main, main)],
                    dst_ref.at[pl.ds(s * main, main)], add=add)
    if rem:
        @pl.when(s == 0)
        def _():
            pltpu.sync_copy(src_ref.at[pl.ds(NS * main, rem)],
                            dst_ref.at[pl.ds(NS * main, rem)], add=add)


@functools.lru_cache(maxsize=None)
def _make_degree_kernel(EP, NPAD):
    blocks = EP // (NC * NS * BLK)

    @functools.partial(
        pl.kernel,
        mesh=_mesh(),
        out_type=jax.ShapeDtypeStruct((NC, 1, NPAD), F32),
        scratch_types=[
            pltpu.VMEM((BLK,), jnp.int32),
            pltpu.VMEM((BLK,), F32),
            pltpu.VMEM_SHARED((NPAD,), F32),
        ],
    )
    def deg_kernel(dst_hbm, zeros_hbm, ones_hbm, out_hbm, didx, ones_v, acc):
        c = lax.axis_index("c")
        s = lax.axis_index("s")

        @pl.when(s == 0)
        def _():
            pltpu.sync_copy(zeros_hbm, acc)

        pltpu.sync_copy(ones_hbm, ones_v)
        plsc.subcore_barrier()
        base = (c * NS + s) * blocks

        def body(j, carry):
            off = pl.multiple_of((base + j) * BLK, BLK)
            pltpu.sync_copy(dst_hbm.at[pl.ds(off, BLK)], didx)
            pltpu.sync_copy(ones_v, acc.at[didx], add=True)
            return carry

        lax.fori_loop(0, blocks, body, 0)
        plsc.subcore_barrier()

        @pl.when(s == 0)
        def _():
            pltpu.sync_copy(acc, out_hbm.at[c, 0])

    return deg_kernel


@functools.lru_cache(maxsize=None)
def _make_agg_kernel(n_chunks, EP, NPAD, N):
    blocks = EP // (NS * BLK)
    chunks_per_core = n_chunks // NC

    NBUF = 2
    PK = BLK // 2

    @functools.partial(
        pl.kernel,
        mesh=_mesh(),
        out_type=jax.ShapeDtypeStruct((n_chunks, N, CW), F32),
        scratch_types=[
            pltpu.VMEM((blocks * PK,), jnp.int32),
            pltpu.VMEM((blocks * PK,), jnp.int32),
            [pltpu.VMEM((BLK,), jnp.int32)] * NBUF,
            [pltpu.VMEM((BLK,), jnp.int32)] * NBUF,
            [pltpu.VMEM((BLK, CW), F32)] * NBUF,
            [pltpu.SemaphoreType.DMA] * NBUF,
            pltpu.VMEM_SHARED((NPAD, CW), F32),
        ],
    )
    def agg_kernel(xs_hbm, spk_hbm, dpk_hbm, out_hbm,
                   spk, dpk, sstag, dstag, rows, sems, acc):
        c = lax.axis_index("c")
        s = lax.axis_index("s")
        pltpu.sync_copy(spk_hbm.at[pl.ds(s * blocks * PK, blocks * PK)], spk)
        pltpu.sync_copy(dpk_hbm.at[pl.ds(s * blocks * PK, blocks * PK)], dpk)

        def unpack(pk, stag, blk):
            for u in range(PK // 16):
                w = pk[pl.ds(blk * PK + u * 16, 16)]
                stag[pl.ds(u * 16, 16)] = w & jnp.int32(0xFFFF)
                stag[pl.ds(PK + u * 16, 16)] = lax.shift_right_logical(w, 16)

        for cc in range(chunks_per_core):
            chunk = c * chunks_per_core + cc
            _copy_rows(xs_hbm.at[chunk], acc, s, N)
            plsc.subcore_barrier()

            def body(j, carry):
                cps = []
                for b in range(NBUF):
                    unpack(spk, sstag[b], NBUF * j + b)
                    cps.append(pltpu.async_copy(
                        xs_hbm.at[chunk].at[sstag[b]], rows[b], sems[b]))
                for b in range(NBUF):
                    unpack(dpk, dstag[b], NBUF * j + b)
                for b in range(NBUF):
                    cps[b].wait()
                    pltpu.sync_copy(rows[b], acc.at[dstag[b]], add=True)
                return carry

            lax.fori_loop(0, blocks // NBUF, body, 0)
            plsc.subcore_barrier()
            _copy_rows(acc, out_hbm.at[chunk], s, N)
            plsc.subcore_barrier()

    return agg_kernel


def _prescale_kernel(x, indeg2, N, DIN, NPAD):
    TN = 1000
    n_chunks = DIN // CW
    ind0 = indeg2[0, 0, :N][:, None]
    ind1 = indeg2[1, 0, :N][:, None]

    def body(ind0_ref, ind1_ref, x_ref, xs_ref, dinv_ref):
        deg = ind0_ref[...] + ind1_ref[...] + 1.0
        dinv = lax.rsqrt(deg)
        dinv_ref[...] = dinv
        xs_ref[...] = (dinv * x_ref[...])[None]

    return pl.pallas_call(
        body,
        grid=(n_chunks, N // TN),
        in_specs=[
            pl.BlockSpec((TN, 1), lambda c, i: (i, 0)),
            pl.BlockSpec((TN, 1), lambda c, i: (i, 0)),
            pl.BlockSpec((TN, CW), lambda c, i: (i, c)),
        ],
        out_specs=[
            pl.BlockSpec((1, TN, CW), lambda c, i: (c, i, 0)),
            pl.BlockSpec((TN, 1), lambda c, i: (i, 0)),
        ],
        out_shape=[
            jax.ShapeDtypeStruct((n_chunks, NPAD, CW), F32),
            jax.ShapeDtypeStruct((N, 1), F32),
        ],
    )(ind0, ind1, x)


def _mm_relu_scale_kernel(y, dinv, W, b, N, NPAD):
    TN = 1000
    KC = y.shape[0]
    DO = W.shape[1]
    JC = DO // CW
    b2d = b.reshape(1, DO)

    def body(y_ref, dinv_ref, w_ref, b_ref, out_ref):
        k = pl.program_id(2)
        part = jnp.dot(dinv_ref[...] * y_ref[0], w_ref[...],
                       preferred_element_type=F32)

        @pl.when(k == 0)
        def _():
            out_ref[...] = part[None]

        @pl.when(k > 0)
        def _():
            out_ref[...] += part[None]

        @pl.when(k == KC - 1)
        def _():
            z = out_ref[0] + b_ref[...]
            out_ref[...] = (dinv_ref[...] * jnp.maximum(z, 0.0))[None]

    return pl.pallas_call(
        body,
        grid=(N // TN, JC, KC),
        in_specs=[
            pl.BlockSpec((1, TN, CW), lambda i, j, k: (k, i, 0)),
            pl.BlockSpec((TN, 1), lambda i, j, k: (i, 0)),
            pl.BlockSpec((CW, CW), lambda i, j, k: (k, j)),
            pl.BlockSpec((1, CW), lambda i, j, k: (0, j)),
        ],
        out_specs=pl.BlockSpec((1, TN, CW), lambda i, j, k: (j, i, 0)),
        out_shape=jax.ShapeDtypeStruct((JC, NPAD, CW), F32),
    )(y, dinv, W, b2d)


def _final_kernel(y, dinv, W, b, fcW, fcb, batchf, N, G):
    TN = 1000
    KC = y.shape[0]
    DO = W.shape[1]
    JC = DO // CW
    b2d = b.reshape(1, DO)
    fcb2d = fcb.reshape(1, 1)
    NI = N // TN

    def body(y_ref, dinv_ref, w_ref, b_ref, fcw_ref, fcb_ref, batch_ref,
             pooled_ref, counts_ref, out_ref, zacc):
        i = pl.program_id(0)
        j = pl.program_id(1)
        k = pl.program_id(2)
        part = jnp.dot(dinv_ref[...] * y_ref[0], w_ref[...],
                       preferred_element_type=F32)

        @pl.when(k == 0)
        def _():
            zacc[...] = part

        @pl.when(k > 0)
        def _():
            zacc[...] += part

        @pl.when(k == KC - 1)
        def _():
            h = jnp.maximum(zacc[...] + b_ref[...], 0.0)
            s = jnp.dot(h, fcw_ref[...], preferred_element_type=F32)
            gids = lax.broadcasted_iota(jnp.int32, (1, G), 1).astype(F32)
            onehot = (batch_ref[...] == gids).astype(F32)
            ppart = lax.dot_general(s, onehot, (((0,), (0,)), ((), ())))

            @pl.when(jnp.logical_and(i == 0, j == 0))
            def _():
                pooled_ref[...] = ppart

            @pl.when(jnp.logical_or(i > 0, j > 0))
            def _():
                pooled_ref[...] += ppart

            @pl.when(j == 0)
            def _():
                cpart = jnp.sum(onehot, axis=0, keepdims=True)

                @pl.when(i == 0)
                def _():
                    counts_ref[...] = cpart

                @pl.when(i > 0)
                def _():
                    counts_ref[...] += cpart

            @pl.when(jnp.logical_and(i == NI - 1, j == JC - 1))
            def _():
                out_ref[...] = (pooled_ref[...]
                                / jnp.maximum(counts_ref[...], 1.0)
                                + fcb_ref[...])

    pooled, counts, out = pl.pallas_call(
        body,
        grid=(NI, JC, KC),
        in_specs=[
            pl.BlockSpec((1, TN, CW), lambda i, j, k: (k, i, 0)),
            pl.BlockSpec((TN, 1), lambda i, j, k: (i, 0)),
            pl.BlockSpec((CW, CW), lambda i, j, k: (k, j)),
            pl.BlockSpec((1, CW), lambda i, j, k: (0, j)),
            pl.BlockSpec((CW, 1), lambda i, j, k: (j, 0)),
            pl.BlockSpec((1, 1), lambda i, j, k: (0, 0)),
            pl.BlockSpec((TN, 1), lambda i, j, k: (i, 0)),
        ],
        out_specs=[
            pl.BlockSpec((1, G), lambda i, j, k: (0, 0)),
            pl.BlockSpec((1, G), lambda i, j, k: (0, 0)),
            pl.BlockSpec((1, G), lambda i, j, k: (0, 0)),
        ],
        out_shape=[
            jax.ShapeDtypeStruct((1, G), F32),
            jax.ShapeDtypeStruct((1, G), F32),
            jax.ShapeDtypeStruct((1, G), F32),
        ],
        scratch_shapes=[pltpu.VMEM((TN, CW), F32)],
    )(y, dinv, W, b2d, fcW, fcb2d, batchf)
    del pooled, counts
    return out


def kernel(x, edge_index, batch, W1, b1, W2, b2, fcW, fcb):
    N, DIN = x.shape
    DH = W1.shape[1]
    G = 64
    NPAD = N + 8
    E = edge_index.shape[1]

    step = NC * NS * BLK
    EP = ((E + step - 1) // step) * step
    pad = EP - E
    src = jnp.concatenate([edge_index[0], jnp.full((pad,), N, jnp.int32)])
    dst = jnp.concatenate([edge_index[1], jnp.full((pad,), N, jnp.int32)])

    deg_k = _make_degree_kernel(EP, NPAD)
    indeg2 = deg_k(dst, jnp.zeros((NPAD,), F32), jnp.ones((BLK,), F32))

    xs1, dinv = _prescale_kernel(x, indeg2, N, DIN, NPAD)

    def pack_idx(a):
        b2 = a.reshape(-1, 2, BLK // 2)
        return (b2[:, 0, :] | (b2[:, 1, :] << 16)).reshape(-1)

    spk = pack_idx(src)
    dpk = pack_idx(dst)

    agg1 = _make_agg_kernel(DIN // CW, EP, NPAD, N)
    y1 = agg1(xs1, spk, dpk)

    xs2 = _mm_relu_scale_kernel(y1, dinv, W1, b1, N, NPAD)

    agg2 = _make_agg_kernel(DH // CW, EP, NPAD, N)
    y2 = agg2(xs2, spk, dpk)

    batchf = batch.astype(F32).reshape(N, 1)
    out = _final_kernel(y2, dinv, W2, b2, fcW, fcb, batchf, N, G)
    return out.reshape(G, 1)


# speedup vs baseline: 1.0571x; 1.0571x over previous
import functools

import jax
import jax.numpy as jnp
from jax import lax
from jax.experimental import pallas as pl
from jax.experimental.pallas import tpu as pltpu
from jax.experimental.pallas import tpu_sc as plsc

F32 = jnp.float32
NC = 2
NS = 16
BLK = 128
CW = 128


def _mesh():
    return plsc.VectorSubcoreMesh(core_axis_name="c", subcore_axis_name="s",
                                  num_cores=NC, num_subcores=NS)


def _copy_rows(src_ref, dst_ref, s, n_rows, add=False):
    main = (n_rows // NS) & ~7
    rem = n_rows - NS * main
    pltpu.sync_copy(src_ref.at[pl.ds(s * main, main)],
                    dst_ref.at[pl.ds(s * main, main)], add=add)
    if rem:
        @pl.when(s == 0)
        def _():
            pltpu.sync_copy(src_ref.at[pl.ds(NS * main, rem)],
                            dst_ref.at[pl.ds(NS * main, rem)], add=add)


@functools.lru_cache(maxsize=None)
def _make_degree_kernel(EP, NPAD):
    blocks = EP // (NC * NS * BLK)

    @functools.partial(
        pl.kernel,
        mesh=_mesh(),
        out_type=jax.ShapeDtypeStruct((NC, 1, NPAD), F32),
        scratch_types=[
            pltpu.VMEM((BLK,), jnp.int32),
            pltpu.VMEM((BLK,), F32),
            pltpu.VMEM_SHARED((NPAD,), F32),
        ],
    )
    def deg_kernel(dst_hbm, zeros_hbm, ones_hbm, out_hbm, didx, ones_v, acc):
        c = lax.axis_index("c")
        s = lax.axis_index("s")

        @pl.when(s == 0)
        def _():
            pltpu.sync_copy(zeros_hbm, acc)

        pltpu.sync_copy(ones_hbm, ones_v)
        plsc.subcore_barrier()
        base = (c * NS + s) * blocks

        def body(j, carry):
            off = pl.multiple_of((base + j) * BLK, BLK)
            pltpu.sync_copy(dst_hbm.at[pl.ds(off, BLK)], didx)
            pltpu.sync_copy(ones_v, acc.at[didx], add=True)
            return carry

        lax.fori_loop(0, blocks, body, 0)
        plsc.subcore_barrier()

        @pl.when(s == 0)
        def _():
            pltpu.sync_copy(acc, out_hbm.at[c, 0])

    return deg_kernel


@functools.lru_cache(maxsize=None)
def _make_agg_kernel(n_chunks, EP, NPAD, N):
    ABLK = 64
    NBUF = 4
    PK = ABLK // 2
    blocks = EP // (NS * ABLK)
    chunks_per_core = n_chunks // NC

    @functools.partial(
        pl.kernel,
        mesh=_mesh(),
        out_type=jax.ShapeDtypeStruct((n_chunks, N, CW), F32),
        scratch_types=[
            pltpu.VMEM((blocks * PK,), jnp.int32),
            pltpu.VMEM((blocks * PK,), jnp.int32),
            [pltpu.VMEM((ABLK,), jnp.int32)] * NBUF,
            [pltpu.VMEM((ABLK,), jnp.int32)] * NBUF,
            [pltpu.VMEM((ABLK, CW), F32)] * NBUF,
            [pltpu.SemaphoreType.DMA] * NBUF,
            [pltpu.SemaphoreType.DMA] * NBUF,
            pltpu.VMEM_SHARED((NPAD, CW), F32),
        ],
    )
    def agg_kernel(xs_hbm, spk_hbm, dpk_hbm, out_hbm,
                   spk, dpk, sstag, dstag, rows, gsems, ssems, acc):
        c = lax.axis_index("c")
        s = lax.axis_index("s")
        pltpu.sync_copy(spk_hbm.at[pl.ds(s * blocks * PK, blocks * PK)], spk)
        pltpu.sync_copy(dpk_hbm.at[pl.ds(s * blocks * PK, blocks * PK)], dpk)

        def unpack(pk, stag, blk):
            for u in range(PK // 16):
                w = pk[pl.ds(blk * PK + u * 16, 16)]
                stag[pl.ds(u * 16, 16)] = w & jnp.int32(0xFFFF)
                stag[pl.ds(PK + u * 16, 16)] = lax.shift_right_logical(w, 16)

        for cc in range(chunks_per_core):
            chunk = c * chunks_per_core + cc
            _copy_rows(xs_hbm.at[chunk], acc, s, N)
            plsc.subcore_barrier()

            for b in range(NBUF):
                unpack(spk, sstag[b], b)
                pltpu.async_copy(xs_hbm.at[chunk].at[sstag[b]],
                                 rows[b], gsems[b])

            def body(j, carry):
                for b in range(NBUF):
                    blk = NBUF * j + b
                    pltpu.make_async_copy(xs_hbm.at[chunk].at[sstag[b]],
                                          rows[b], gsems[b]).wait()
                    unpack(dpk, dstag[b], blk)
                    pltpu.async_copy(rows[b], acc.at[dstag[b]], ssems[b],
                                     add=True)
                    nxt = lax.min(blk + NBUF, blocks - 1)
                    unpack(spk, sstag[b], nxt)
                    pltpu.make_async_copy(rows[b], acc.at[dstag[b]],
                                          ssems[b]).wait()
                    pltpu.async_copy(xs_hbm.at[chunk].at[sstag[b]],
                                     rows[b], gsems[b])
                return carry

            lax.fori_loop(0, blocks // NBUF, body, 0)
            for b in range(NBUF):
                pltpu.make_async_copy(xs_hbm.at[chunk].at[sstag[b]],
                                      rows[b], gsems[b]).wait()
            plsc.subcore_barrier()
            _copy_rows(acc, out_hbm.at[chunk], s, N)
            plsc.subcore_barrier()

    return agg_kernel


def _prescale_kernel(x, indeg2, N, DIN, NPAD):
    TN = 1000
    n_chunks = DIN // CW
    ind0 = indeg2[0, 0, :N][:, None]
    ind1 = indeg2[1, 0, :N][:, None]

    def body(ind0_ref, ind1_ref, x_ref, xs_ref, dinv_ref):
        deg = ind0_ref[...] + ind1_ref[...] + 1.0
        dinv = lax.rsqrt(deg)
        dinv_ref[...] = dinv
        xs_ref[...] = (dinv * x_ref[...])[None]

    return pl.pallas_call(
        body,
        grid=(n_chunks, N // TN),
        in_specs=[
            pl.BlockSpec((TN, 1), lambda c, i: (i, 0)),
            pl.BlockSpec((TN, 1), lambda c, i: (i, 0)),
            pl.BlockSpec((TN, CW), lambda c, i: (i, c)),
        ],
        out_specs=[
            pl.BlockSpec((1, TN, CW), lambda c, i: (c, i, 0)),
            pl.BlockSpec((TN, 1), lambda c, i: (i, 0)),
        ],
        out_shape=[
            jax.ShapeDtypeStruct((n_chunks, NPAD, CW), F32),
            jax.ShapeDtypeStruct((N, 1), F32),
        ],
    )(ind0, ind1, x)


def _mm_relu_scale_kernel(y, dinv, W, b, N, NPAD):
    TN = 1000
    KC = y.shape[0]
    DO = W.shape[1]
    JC = DO // CW
    b2d = b.reshape(1, DO)

    def body(y_ref, dinv_ref, w_ref, b_ref, out_ref):
        k = pl.program_id(2)
        part = jnp.dot(dinv_ref[...] * y_ref[0], w_ref[...],
                       preferred_element_type=F32)

        @pl.when(k == 0)
        def _():
            out_ref[...] = part[None]

        @pl.when(k > 0)
        def _():
            out_ref[...] += part[None]

        @pl.when(k == KC - 1)
        def _():
            z = out_ref[0] + b_ref[...]
            out_ref[...] = (dinv_ref[...] * jnp.maximum(z, 0.0))[None]

    return pl.pallas_call(
        body,
        grid=(N // TN, JC, KC),
        in_specs=[
            pl.BlockSpec((1, TN, CW), lambda i, j, k: (k, i, 0)),
            pl.BlockSpec((TN, 1), lambda i, j, k: (i, 0)),
            pl.BlockSpec((CW, CW), lambda i, j, k: (k, j)),
            pl.BlockSpec((1, CW), lambda i, j, k: (0, j)),
        ],
        out_specs=pl.BlockSpec((1, TN, CW), lambda i, j, k: (j, i, 0)),
        out_shape=jax.ShapeDtypeStruct((JC, NPAD, CW), F32),
    )(y, dinv, W, b2d)


def _final_kernel(y, dinv, W, b, fcW, fcb, batchf, N, G):
    TN = 1000
    KC = y.shape[0]
    DO = W.shape[1]
    JC = DO // CW
    b2d = b.reshape(1, DO)
    fcb2d = fcb.reshape(1, 1)
    NI = N // TN

    def body(y_ref, dinv_ref, w_ref, b_ref, fcw_ref, fcb_ref, batch_ref,
             pooled_ref, counts_ref, out_ref, zacc):
        i = pl.program_id(0)
        j = pl.program_id(1)
        k = pl.program_id(2)
        part = jnp.dot(dinv_ref[...] * y_ref[0], w_ref[...],
                       preferred_element_type=F32)

        @pl.when(k == 0)
        def _():
            zacc[...] = part

        @pl.when(k > 0)
        def _():
            zacc[...] += part

        @pl.when(k == KC - 1)
        def _():
            h = jnp.maximum(zacc[...] + b_ref[...], 0.0)
            s = jnp.dot(h, fcw_ref[...], preferred_element_type=F32)
            gids = lax.broadcasted_iota(jnp.int32, (1, G), 1).astype(F32)
            onehot = (batch_ref[...] == gids).astype(F32)
            ppart = lax.dot_general(s, onehot, (((0,), (0,)), ((), ())))

            @pl.when(jnp.logical_and(i == 0, j == 0))
            def _():
                pooled_ref[...] = ppart

            @pl.when(jnp.logical_or(i > 0, j > 0))
            def _():
                pooled_ref[...] += ppart

            @pl.when(j == 0)
            def _():
                cpart = jnp.sum(onehot, axis=0, keepdims=True)

                @pl.when(i == 0)
                def _():
                    counts_ref[...] = cpart

                @pl.when(i > 0)
                def _():
                    counts_ref[...] += cpart

            @pl.when(jnp.logical_and(i == NI - 1, j == JC - 1))
            def _():
                out_ref[...] = (pooled_ref[...]
                                / jnp.maximum(counts_ref[...], 1.0)
                                + fcb_ref[...])

    pooled, counts, out = pl.pallas_call(
        body,
        grid=(NI, JC, KC),
        in_specs=[
            pl.BlockSpec((1, TN, CW), lambda i, j, k: (k, i, 0)),
            pl.BlockSpec((TN, 1), lambda i, j, k: (i, 0)),
            pl.BlockSpec((CW, CW), lambda i, j, k: (k, j)),
            pl.BlockSpec((1, CW), lambda i, j, k: (0, j)),
            pl.BlockSpec((CW, 1), lambda i, j, k: (j, 0)),
            pl.BlockSpec((1, 1), lambda i, j, k: (0, 0)),
            pl.BlockSpec((TN, 1), lambda i, j, k: (i, 0)),
        ],
        out_specs=[
            pl.BlockSpec((1, G), lambda i, j, k: (0, 0)),
            pl.BlockSpec((1, G), lambda i, j, k: (0, 0)),
            pl.BlockSpec((1, G), lambda i, j, k: (0, 0)),
        ],
        out_shape=[
            jax.ShapeDtypeStruct((1, G), F32),
            jax.ShapeDtypeStruct((1, G), F32),
            jax.ShapeDtypeStruct((1, G), F32),
        ],
        scratch_shapes=[pltpu.VMEM((TN, CW), F32)],
    )(y, dinv, W, b2d, fcW, fcb2d, batchf)
    del pooled, counts
    return out


def kernel(x, edge_index, batch, W1, b1, W2, b2, fcW, fcb):
    N, DIN = x.shape
    DH = W1.shape[1]
    G = 64
    NPAD = N + 8
    E = edge_index.shape[1]

    step = NC * NS * BLK
    EP = ((E + step - 1) // step) * step
    pad = EP - E
    src = jnp.concatenate([edge_index[0], jnp.full((pad,), N, jnp.int32)])
    dst = jnp.concatenate([edge_index[1], jnp.full((pad,), N, jnp.int32)])

    deg_k = _make_degree_kernel(EP, NPAD)
    indeg2 = deg_k(dst, jnp.zeros((NPAD,), F32), jnp.ones((BLK,), F32))

    xs1, dinv = _prescale_kernel(x, indeg2, N, DIN, NPAD)

    def pack_idx(a):
        b2 = a.reshape(-1, 2, 32)
        return (b2[:, 0, :] | (b2[:, 1, :] << 16)).reshape(-1)

    spk = pack_idx(src)
    dpk = pack_idx(dst)

    agg1 = _make_agg_kernel(DIN // CW, EP, NPAD, N)
    y1 = agg1(xs1, spk, dpk)

    xs2 = _mm_relu_scale_kernel(y1, dinv, W1, b1, N, NPAD)

    agg2 = _make_agg_kernel(DH // CW, EP, NPAD, N)
    y2 = agg2(xs2, spk, dpk)

    batchf = batch.astype(F32).reshape(N, 1)
    out = _final_kernel(y2, dinv, W2, b2, fcW, fcb, batchf, N, G)
    return out.reshape(G, 1)


# speedup vs baseline: 1.0574x; 1.0002x over previous
import functools

import jax
import jax.numpy as jnp
from jax import lax
from jax.experimental import pallas as pl
from jax.experimental.pallas import tpu as pltpu
from jax.experimental.pallas import tpu_sc as plsc

F32 = jnp.float32
NC = 2
NS = 16
BLK = 128
CW = 128


def _mesh():
    return plsc.VectorSubcoreMesh(core_axis_name="c", subcore_axis_name="s",
                                  num_cores=NC, num_subcores=NS)


def _copy_rows(src_ref, dst_ref, s, n_rows, add=False):
    main = (n_rows // NS) & ~7
    rem = n_rows - NS * main
    pltpu.sync_copy(src_ref.at[pl.ds(s * main, main)],
                    dst_ref.at[pl.ds(s * main, main)], add=add)
    if rem:
        @pl.when(s == 0)
        def _():
            pltpu.sync_copy(src_ref.at[pl.ds(NS * main, rem)],
                            dst_ref.at[pl.ds(NS * main, rem)], add=add)


@functools.lru_cache(maxsize=None)
def _make_degree_kernel(EP, NPAD):
    blocks = EP // (NC * NS * BLK)

    @functools.partial(
        pl.kernel,
        mesh=_mesh(),
        out_type=jax.ShapeDtypeStruct((NC, 1, NPAD), F32),
        scratch_types=[
            pltpu.VMEM((BLK,), jnp.int32),
            pltpu.VMEM((BLK,), F32),
            pltpu.VMEM_SHARED((NPAD,), F32),
        ],
    )
    def deg_kernel(dst_hbm, zeros_hbm, ones_hbm, out_hbm, didx, ones_v, acc):
        c = lax.axis_index("c")
        s = lax.axis_index("s")

        @pl.when(s == 0)
        def _():
            pltpu.sync_copy(zeros_hbm, acc)

        pltpu.sync_copy(ones_hbm, ones_v)
        plsc.subcore_barrier()
        base = (c * NS + s) * blocks

        def body(j, carry):
            off = pl.multiple_of((base + j) * BLK, BLK)
            pltpu.sync_copy(dst_hbm.at[pl.ds(off, BLK)], didx)
            pltpu.sync_copy(ones_v, acc.at[didx], add=True)
            return carry

        lax.fori_loop(0, blocks, body, 0)
        plsc.subcore_barrier()

        @pl.when(s == 0)
        def _():
            pltpu.sync_copy(acc, out_hbm.at[c, 0])

    return deg_kernel


@functools.lru_cache(maxsize=None)
def _make_agg_kernel(n_chunks, EP, NPAD, N):
    ABLK = 64
    NBUF = 4
    PK = ABLK // 2
    blocks = EP // (NS * ABLK)
    chunks_per_core = n_chunks // NC

    @functools.partial(
        pl.kernel,
        mesh=_mesh(),
        out_type=jax.ShapeDtypeStruct((n_chunks, N, CW), F32),
        scratch_types=[
            pltpu.VMEM((blocks * PK,), jnp.int32),
            pltpu.VMEM((blocks * PK,), jnp.int32),
            [pltpu.VMEM((ABLK,), jnp.int32)] * NBUF,
            [pltpu.VMEM((ABLK,), jnp.int32)] * NBUF,
            [pltpu.VMEM((ABLK, CW), F32)] * NBUF,
            [pltpu.SemaphoreType.DMA] * NBUF,
            [pltpu.SemaphoreType.DMA] * NBUF,
            pltpu.VMEM_SHARED((NPAD, CW), F32),
        ],
    )
    def agg_kernel(xs_hbm, spk_hbm, dpk_hbm, out_hbm,
                   spk, dpk, sstag, dstag, rows, gsems, ssems, acc):
        c = lax.axis_index("c")
        s = lax.axis_index("s")
        pltpu.sync_copy(spk_hbm.at[pl.ds(s * blocks * PK, blocks * PK)], spk)
        pltpu.sync_copy(dpk_hbm.at[pl.ds(s * blocks * PK, blocks * PK)], dpk)

        def unpack(pk, stag, blk):
            for u in range(PK // 16):
                w = pk[pl.ds(blk * PK + u * 16, 16)]
                stag[pl.ds(u * 16, 16)] = w & jnp.int32(0xFFFF)
                stag[pl.ds(PK + u * 16, 16)] = lax.shift_right_logical(w, 16)

        for cc in range(chunks_per_core):
            chunk = c * chunks_per_core + cc
            _copy_rows(xs_hbm.at[chunk], acc, s, N)
            plsc.subcore_barrier()

            for b in range(NBUF):
                unpack(spk, sstag[b], b)
                pltpu.async_copy(xs_hbm.at[chunk].at[sstag[b]],
                                 rows[b], gsems[b])

            def body(j, carry):
                for b in range(NBUF):
                    blk = NBUF * j + b
                    unpack(dpk, dstag[b], blk)
                    pltpu.make_async_copy(xs_hbm.at[chunk].at[sstag[b]],
                                          rows[b], gsems[b]).wait()
                    pltpu.async_copy(rows[b], acc.at[dstag[b]], ssems[b],
                                     add=True)
                    nxt = lax.min(blk + NBUF, blocks - 1)
                    unpack(spk, sstag[b], nxt)
                    pltpu.make_async_copy(rows[b], acc.at[dstag[b]],
                                          ssems[b]).wait()
                    pltpu.async_copy(xs_hbm.at[chunk].at[sstag[b]],
                                     rows[b], gsems[b])
                return carry

            lax.fori_loop(0, blocks // NBUF, body, 0)
            for b in range(NBUF):
                pltpu.make_async_copy(xs_hbm.at[chunk].at[sstag[b]],
                                      rows[b], gsems[b]).wait()
            plsc.subcore_barrier()
            _copy_rows(acc, out_hbm.at[chunk], s, N)
            plsc.subcore_barrier()

    return agg_kernel


def _prescale_kernel(x, indeg2, N, DIN, NPAD):
    TN = 1000
    n_chunks = DIN // CW
    ind0 = indeg2[0, 0, :N][:, None]
    ind1 = indeg2[1, 0, :N][:, None]

    def body(ind0_ref, ind1_ref, x_ref, xs_ref, dinv_ref):
        deg = ind0_ref[...] + ind1_ref[...] + 1.0
        dinv = lax.rsqrt(deg)
        dinv_ref[...] = dinv
        xs_ref[...] = (dinv * x_ref[...])[None]

    return pl.pallas_call(
        body,
        grid=(n_chunks, N // TN),
        in_specs=[
            pl.BlockSpec((TN, 1), lambda c, i: (i, 0)),
            pl.BlockSpec((TN, 1), lambda c, i: (i, 0)),
            pl.BlockSpec((TN, CW), lambda c, i: (i, c)),
        ],
        out_specs=[
            pl.BlockSpec((1, TN, CW), lambda c, i: (c, i, 0)),
            pl.BlockSpec((TN, 1), lambda c, i: (i, 0)),
        ],
        out_shape=[
            jax.ShapeDtypeStruct((n_chunks, NPAD, CW), F32),
            jax.ShapeDtypeStruct((N, 1), F32),
        ],
    )(ind0, ind1, x)


def _mm_relu_scale_kernel(y, dinv, W, b, N, NPAD):
    TN = 1000
    KC = y.shape[0]
    DO = W.shape[1]
    JC = DO // CW
    b2d = b.reshape(1, DO)

    def body(y_ref, dinv_ref, w_ref, b_ref, out_ref):
        k = pl.program_id(2)
        part = jnp.dot(dinv_ref[...] * y_ref[0], w_ref[...],
                       preferred_element_type=F32)

        @pl.when(k == 0)
        def _():
            out_ref[...] = part[None]

        @pl.when(k > 0)
        def _():
            out_ref[...] += part[None]

        @pl.when(k == KC - 1)
        def _():
            z = out_ref[0] + b_ref[...]
            out_ref[...] = (dinv_ref[...] * jnp.maximum(z, 0.0))[None]

    return pl.pallas_call(
        body,
        grid=(N // TN, JC, KC),
        in_specs=[
            pl.BlockSpec((1, TN, CW), lambda i, j, k: (k, i, 0)),
            pl.BlockSpec((TN, 1), lambda i, j, k: (i, 0)),
            pl.BlockSpec((CW, CW), lambda i, j, k: (k, j)),
            pl.BlockSpec((1, CW), lambda i, j, k: (0, j)),
        ],
        out_specs=pl.BlockSpec((1, TN, CW), lambda i, j, k: (j, i, 0)),
        out_shape=jax.ShapeDtypeStruct((JC, NPAD, CW), F32),
    )(y, dinv, W, b2d)


def _final_kernel(y, dinv, W, b, fcW, fcb, batchf, N, G):
    TN = 1000
    KC = y.shape[0]
    DO = W.shape[1]
    JC = DO // CW
    b2d = b.reshape(1, DO)
    fcb2d = fcb.reshape(1, 1)
    NI = N // TN

    def body(y_ref, dinv_ref, w_ref, b_ref, fcw_ref, fcb_ref, batch_ref,
             pooled_ref, counts_ref, out_ref, zacc):
        i = pl.program_id(0)
        j = pl.program_id(1)
        k = pl.program_id(2)
        part = jnp.dot(dinv_ref[...] * y_ref[0], w_ref[...],
                       preferred_element_type=F32)

        @pl.when(k == 0)
        def _():
            zacc[...] = part

        @pl.when(k > 0)
        def _():
            zacc[...] += part

        @pl.when(k == KC - 1)
        def _():
            h = jnp.maximum(zacc[...] + b_ref[...], 0.0)
            s = jnp.dot(h, fcw_ref[...], preferred_element_type=F32)
            gids = lax.broadcasted_iota(jnp.int32, (1, G), 1).astype(F32)
            onehot = (batch_ref[...] == gids).astype(F32)
            ppart = lax.dot_general(s, onehot, (((0,), (0,)), ((), ())))

            @pl.when(jnp.logical_and(i == 0, j == 0))
            def _():
                pooled_ref[...] = ppart

            @pl.when(jnp.logical_or(i > 0, j > 0))
            def _():
                pooled_ref[...] += ppart

            @pl.when(j == 0)
            def _():
                cpart = jnp.sum(onehot, axis=0, keepdims=True)

                @pl.when(i == 0)
                def _():
                    counts_ref[...] = cpart

                @pl.when(i > 0)
                def _():
                    counts_ref[...] += cpart

            @pl.when(jnp.logical_and(i == NI - 1, j == JC - 1))
            def _():
                out_ref[...] = (pooled_ref[...]
                                / jnp.maximum(counts_ref[...], 1.0)
                                + fcb_ref[...])

    pooled, counts, out = pl.pallas_call(
        body,
        grid=(NI, JC, KC),
        in_specs=[
            pl.BlockSpec((1, TN, CW), lambda i, j, k: (k, i, 0)),
            pl.BlockSpec((TN, 1), lambda i, j, k: (i, 0)),
            pl.BlockSpec((CW, CW), lambda i, j, k: (k, j)),
            pl.BlockSpec((1, CW), lambda i, j, k: (0, j)),
            pl.BlockSpec((CW, 1), lambda i, j, k: (j, 0)),
            pl.BlockSpec((1, 1), lambda i, j, k: (0, 0)),
            pl.BlockSpec((TN, 1), lambda i, j, k: (i, 0)),
        ],
        out_specs=[
            pl.BlockSpec((1, G), lambda i, j, k: (0, 0)),
            pl.BlockSpec((1, G), lambda i, j, k: (0, 0)),
            pl.BlockSpec((1, G), lambda i, j, k: (0, 0)),
        ],
        out_shape=[
            jax.ShapeDtypeStruct((1, G), F32),
            jax.ShapeDtypeStruct((1, G), F32),
            jax.ShapeDtypeStruct((1, G), F32),
        ],
        scratch_shapes=[pltpu.VMEM((TN, CW), F32)],
    )(y, dinv, W, b2d, fcW, fcb2d, batchf)
    del pooled, counts
    return out


def kernel(x, edge_index, batch, W1, b1, W2, b2, fcW, fcb):
    N, DIN = x.shape
    DH = W1.shape[1]
    G = 64
    NPAD = N + 8
    E = edge_index.shape[1]

    step = NC * NS * BLK
    EP = ((E + step - 1) // step) * step
    pad = EP - E
    src = jnp.concatenate([edge_index[0], jnp.full((pad,), N, jnp.int32)])
    dst = jnp.concatenate([edge_index[1], jnp.full((pad,), N, jnp.int32)])

    deg_k = _make_degree_kernel(EP, NPAD)
    indeg2 = deg_k(dst, jnp.zeros((NPAD,), F32), jnp.ones((BLK,), F32))

    xs1, dinv = _prescale_kernel(x, indeg2, N, DIN, NPAD)

    def pack_idx(a):
        b2 = a.reshape(-1, 2, 32)
        return (b2[:, 0, :] | (b2[:, 1, :] << 16)).reshape(-1)

    spk = pack_idx(src)
    dpk = pack_idx(dst)

    agg1 = _make_agg_kernel(DIN // CW, EP, NPAD, N)
    y1 = agg1(xs1, spk, dpk)

    xs2 = _mm_relu_scale_kernel(y1, dinv, W1, b1, N, NPAD)

    agg2 = _make_agg_kernel(DH // CW, EP, NPAD, N)
    y2 = agg2(xs2, spk, dpk)

    batchf = batch.astype(F32).reshape(N, 1)
    out = _final_kernel(y2, dinv, W2, b2, fcW, fcb, batchf, N, G)
    return out.reshape(G, 1)


# speedup vs baseline: 1.1329x; 1.0715x over previous
import functools

import jax
import jax.numpy as jnp
from jax import lax
from jax.experimental import pallas as pl
from jax.experimental.pallas import tpu as pltpu
from jax.experimental.pallas import tpu_sc as plsc

F32 = jnp.float32
NC = 2
NS = 16
BLK = 128
CW = 128


def _mesh():
    return plsc.VectorSubcoreMesh(core_axis_name="c", subcore_axis_name="s",
                                  num_cores=NC, num_subcores=NS)


def _copy_rows(src_ref, dst_ref, s, n_rows, add=False):
    main = (n_rows // NS) & ~7
    rem = n_rows - NS * main
    pltpu.sync_copy(src_ref.at[pl.ds(s * main, main)],
                    dst_ref.at[pl.ds(s * main, main)], add=add)
    if rem:
        @pl.when(s == 0)
        def _():
            pltpu.sync_copy(src_ref.at[pl.ds(NS * main, rem)],
                            dst_ref.at[pl.ds(NS * main, rem)], add=add)


@functools.lru_cache(maxsize=None)
def _make_degree_kernel(EP, NPAD):
    blocks = EP // (NC * NS * BLK)

    @functools.partial(
        pl.kernel,
        mesh=_mesh(),
        out_type=jax.ShapeDtypeStruct((NC, 1, NPAD), F32),
        scratch_types=[
            pltpu.VMEM((BLK,), jnp.int32),
            pltpu.VMEM((BLK,), F32),
            pltpu.VMEM_SHARED((NPAD,), F32),
        ],
    )
    def deg_kernel(dst_hbm, zeros_hbm, ones_hbm, out_hbm, didx, ones_v, acc):
        c = lax.axis_index("c")
        s = lax.axis_index("s")

        @pl.when(s == 0)
        def _():
            pltpu.sync_copy(zeros_hbm, acc)

        pltpu.sync_copy(ones_hbm, ones_v)
        plsc.subcore_barrier()
        base = (c * NS + s) * blocks

        def body(j, carry):
            off = pl.multiple_of((base + j) * BLK, BLK)
            pltpu.sync_copy(dst_hbm.at[pl.ds(off, BLK)], didx)
            pltpu.sync_copy(ones_v, acc.at[didx], add=True)
            return carry

        lax.fori_loop(0, blocks, body, 0)
        plsc.subcore_barrier()

        @pl.when(s == 0)
        def _():
            pltpu.sync_copy(acc, out_hbm.at[c, 0])

    return deg_kernel


@functools.lru_cache(maxsize=None)
def _make_agg_kernel(n_chunks, EP, NPAD, N):
    ABLK = 64
    NBUF = 4
    PK = ABLK // 2
    blocks = EP // (NS * ABLK)
    chunks_per_core = n_chunks // NC

    @functools.partial(
        pl.kernel,
        mesh=_mesh(),
        out_type=jax.ShapeDtypeStruct((n_chunks, N, CW), F32),
        scratch_types=[
            pltpu.VMEM((blocks * PK,), jnp.int32),
            pltpu.VMEM((blocks * PK,), jnp.int32),
            [pltpu.VMEM((ABLK,), jnp.int32)] * NBUF,
            [pltpu.VMEM((ABLK,), jnp.int32)] * NBUF,
            [pltpu.VMEM((ABLK, CW), F32)] * NBUF,
            [pltpu.SemaphoreType.DMA] * NBUF,
            [pltpu.SemaphoreType.DMA] * NBUF,
            pltpu.VMEM_SHARED((NPAD, CW), F32),
        ],
    )
    def agg_kernel(xs_hbm, spk_hbm, dpk_hbm, out_hbm,
                   spk, dpk, sstag, dstag, rows, gsems, ssems, acc):
        c = lax.axis_index("c")
        s = lax.axis_index("s")
        pltpu.sync_copy(spk_hbm.at[pl.ds(s * blocks * PK, blocks * PK)], spk)
        pltpu.sync_copy(dpk_hbm.at[pl.ds(s * blocks * PK, blocks * PK)], dpk)

        def unpack(pk, stag, blk):
            for u in range(PK // 16):
                w = pk[pl.ds(blk * PK + u * 16, 16)]
                stag[pl.ds(u * 16, 16)] = w & jnp.int32(0xFFFF)
                stag[pl.ds(PK + u * 16, 16)] = lax.shift_right_logical(w, 16)

        for cc in range(chunks_per_core):
            chunk = c * chunks_per_core + cc
            _copy_rows(xs_hbm.at[chunk], acc, s, N)
            plsc.subcore_barrier()

            for b in range(NBUF):
                unpack(spk, sstag[b], b)
                pltpu.async_copy(xs_hbm.at[chunk].at[sstag[b]],
                                 rows[b], gsems[b])

            def body(j, carry):
                for b in range(NBUF):
                    blk = NBUF * j + b
                    unpack(dpk, dstag[b], blk)
                    pltpu.make_async_copy(xs_hbm.at[chunk].at[sstag[b]],
                                          rows[b], gsems[b]).wait()
                    pltpu.async_copy(rows[b], acc.at[dstag[b]], ssems[b],
                                     add=True)
                    nxt = lax.min(blk + NBUF, blocks - 1)
                    unpack(spk, sstag[b], nxt)
                    pltpu.make_async_copy(rows[b], acc.at[dstag[b]],
                                          ssems[b]).wait()
                    pltpu.async_copy(xs_hbm.at[chunk].at[sstag[b]],
                                     rows[b], gsems[b])
                return carry

            lax.fori_loop(0, blocks // NBUF, body, 0)
            for b in range(NBUF):
                pltpu.make_async_copy(xs_hbm.at[chunk].at[sstag[b]],
                                      rows[b], gsems[b]).wait()
            plsc.subcore_barrier()
            _copy_rows(acc, out_hbm.at[chunk], s, N)
            plsc.subcore_barrier()

    return agg_kernel


def _prescale_kernel(x, indeg2, N, DIN, NPAD):
    TN = 2000
    n_chunks = DIN // CW
    ind0 = indeg2[0, 0, :N][:, None]
    ind1 = indeg2[1, 0, :N][:, None]

    def body(ind0_ref, ind1_ref, x_ref, xs_ref, dinv_ref):
        deg = ind0_ref[...] + ind1_ref[...] + 1.0
        dinv = lax.rsqrt(deg)
        dinv_ref[...] = dinv
        xs_ref[...] = (dinv * x_ref[...])[None]

    return pl.pallas_call(
        body,
        grid=(n_chunks, N // TN),
        in_specs=[
            pl.BlockSpec((TN, 1), lambda c, i: (i, 0)),
            pl.BlockSpec((TN, 1), lambda c, i: (i, 0)),
            pl.BlockSpec((TN, CW), lambda c, i: (i, c)),
        ],
        out_specs=[
            pl.BlockSpec((1, TN, CW), lambda c, i: (c, i, 0)),
            pl.BlockSpec((TN, 1), lambda c, i: (i, 0)),
        ],
        out_shape=[
            jax.ShapeDtypeStruct((n_chunks, NPAD, CW), F32),
            jax.ShapeDtypeStruct((N, 1), F32),
        ],
    )(ind0, ind1, x)


def _mm_relu_scale_kernel(y, dinv, W, b, N, NPAD):
    TN = 2000
    KC = y.shape[0]
    DO = W.shape[1]
    JC = DO // CW
    b2d = b.reshape(1, DO)

    def body(y_ref, dinv_ref, w_ref, b_ref, out_ref):
        k = pl.program_id(2)
        part = jnp.dot(dinv_ref[...] * y_ref[0], w_ref[...],
                       preferred_element_type=F32)

        @pl.when(k == 0)
        def _():
            out_ref[...] = part[None]

        @pl.when(k > 0)
        def _():
            out_ref[...] += part[None]

        @pl.when(k == KC - 1)
        def _():
            z = out_ref[0] + b_ref[...]
            out_ref[...] = (dinv_ref[...] * jnp.maximum(z, 0.0))[None]

    return pl.pallas_call(
        body,
        grid=(N // TN, JC, KC),
        in_specs=[
            pl.BlockSpec((1, TN, CW), lambda i, j, k: (k, i, 0)),
            pl.BlockSpec((TN, 1), lambda i, j, k: (i, 0)),
            pl.BlockSpec((CW, CW), lambda i, j, k: (k, j)),
            pl.BlockSpec((1, CW), lambda i, j, k: (0, j)),
        ],
        out_specs=pl.BlockSpec((1, TN, CW), lambda i, j, k: (j, i, 0)),
        out_shape=jax.ShapeDtypeStruct((JC, NPAD, CW), F32),
    )(y, dinv, W, b2d)


def _final_kernel(y, dinv, W, b, fcW, fcb, batchf, N, G):
    TN = 2000
    KC = y.shape[0]
    DO = W.shape[1]
    JC = DO // CW
    b2d = b.reshape(1, DO)
    fcb2d = fcb.reshape(1, 1)
    NI = N // TN

    def body(y_ref, dinv_ref, w_ref, b_ref, fcw_ref, fcb_ref, batch_ref,
             pooled_ref, counts_ref, out_ref, zacc):
        i = pl.program_id(0)
        j = pl.program_id(1)
        k = pl.program_id(2)
        part = jnp.dot(dinv_ref[...] * y_ref[0], w_ref[...],
                       preferred_element_type=F32)

        @pl.when(k == 0)
        def _():
            zacc[...] = part

        @pl.when(k > 0)
        def _():
            zacc[...] += part

        @pl.when(k == KC - 1)
        def _():
            h = jnp.maximum(zacc[...] + b_ref[...], 0.0)
            s = jnp.dot(h, fcw_ref[...], preferred_element_type=F32)
            gids = lax.broadcasted_iota(jnp.int32, (1, G), 1).astype(F32)
            onehot = (batch_ref[...] == gids).astype(F32)
            ppart = lax.dot_general(s, onehot, (((0,), (0,)), ((), ())))

            @pl.when(jnp.logical_and(i == 0, j == 0))
            def _():
                pooled_ref[...] = ppart

            @pl.when(jnp.logical_or(i > 0, j > 0))
            def _():
                pooled_ref[...] += ppart

            @pl.when(j == 0)
            def _():
                cpart = jnp.sum(onehot, axis=0, keepdims=True)

                @pl.when(i == 0)
                def _():
                    counts_ref[...] = cpart

                @pl.when(i > 0)
                def _():
                    counts_ref[...] += cpart

            @pl.when(jnp.logical_and(i == NI - 1, j == JC - 1))
            def _():
                out_ref[...] = (pooled_ref[...]
                                / jnp.maximum(counts_ref[...], 1.0)
                                + fcb_ref[...])

    pooled, counts, out = pl.pallas_call(
        body,
        grid=(NI, JC, KC),
        in_specs=[
            pl.BlockSpec((1, TN, CW), lambda i, j, k: (k, i, 0)),
            pl.BlockSpec((TN, 1), lambda i, j, k: (i, 0)),
            pl.BlockSpec((CW, CW), lambda i, j, k: (k, j)),
            pl.BlockSpec((1, CW), lambda i, j, k: (0, j)),
            pl.BlockSpec((CW, 1), lambda i, j, k: (j, 0)),
            pl.BlockSpec((1, 1), lambda i, j, k: (0, 0)),
            pl.BlockSpec((TN, 1), lambda i, j, k: (i, 0)),
        ],
        out_specs=[
            pl.BlockSpec((1, G), lambda i, j, k: (0, 0)),
            pl.BlockSpec((1, G), lambda i, j, k: (0, 0)),
            pl.BlockSpec((1, G), lambda i, j, k: (0, 0)),
        ],
        out_shape=[
            jax.ShapeDtypeStruct((1, G), F32),
            jax.ShapeDtypeStruct((1, G), F32),
            jax.ShapeDtypeStruct((1, G), F32),
        ],
        scratch_shapes=[pltpu.VMEM((TN, CW), F32)],
    )(y, dinv, W, b2d, fcW, fcb2d, batchf)
    del pooled, counts
    return out


def kernel(x, edge_index, batch, W1, b1, W2, b2, fcW, fcb):
    N, DIN = x.shape
    DH = W1.shape[1]
    G = 64
    NPAD = N + 8
    E = edge_index.shape[1]

    step = NC * NS * BLK
    EP = ((E + step - 1) // step) * step
    pad = EP - E
    src = jnp.concatenate([edge_index[0], jnp.full((pad,), N, jnp.int32)])
    dst = jnp.concatenate([edge_index[1], jnp.full((pad,), N, jnp.int32)])

    deg_k = _make_degree_kernel(EP, NPAD)
    indeg2 = deg_k(dst, jnp.zeros((NPAD,), F32), jnp.ones((BLK,), F32))

    xs1, dinv = _prescale_kernel(x, indeg2, N, DIN, NPAD)

    def pack_idx(a):
        b2 = a.reshape(-1, 2, 32)
        return (b2[:, 0, :] | (b2[:, 1, :] << 16)).reshape(-1)

    spk = pack_idx(src)
    dpk = pack_idx(dst)

    agg1 = _make_agg_kernel(DIN // CW, EP, NPAD, N)
    y1 = agg1(xs1, spk, dpk)

    xs2 = _mm_relu_scale_kernel(y1, dinv, W1, b1, N, NPAD)

    agg2 = _make_agg_kernel(DH // CW, EP, NPAD, N)
    y2 = agg2(xs2, spk, dpk)

    batchf = batch.astype(F32).reshape(N, 1)
    out = _final_kernel(y2, dinv, W2, b2, fcW, fcb, batchf, N, G)
    return out.reshape(G, 1)
